# Initial kernel scaffold; baseline (speedup 1.0000x reference)
import jax
import jax.numpy as jnp
from jax.experimental import pallas as pl


def kernel(node_features, node_num_tensor, weight):
    raise NotImplementedError("write your pallas kernel here")


# speedup vs baseline: 2.8783x; 2.8783x over previous
import jax
import jax.numpy as jnp
import numpy as np
from jax.experimental import pallas as pl
from jax.experimental.pallas import tpu as pltpu

WP = 6
WF = 6


def _edge_att_kernel(lens_ref, nf_ref, w_ref, out_ref):
    b = pl.program_id(0)
    nf = nf_ref[0]
    w = w_ref[...]
    att = jnp.dot(nf, w.T, preferred_element_type=jnp.float32)
    scores = jnp.dot(nf, att.T, preferred_element_type=jnp.float32)
    scores = scores * np.float32(1.0 / np.sqrt(200.0))
    L = scores.shape[0]
    j = jax.lax.broadcasted_iota(jnp.int32, (L, L), 0)
    k = jax.lax.broadcasted_iota(jnp.int32, (L, L), 1)
    n = lens_ref[b]
    mask = (k >= j - WP) & (k <= j + WF) & (k < n) & (j < n)
    masked = jnp.where(mask, scores, jnp.float32(-1e9))
    m = jnp.max(masked, axis=1, keepdims=True)
    e = jnp.exp(masked - m)
    p = e / jnp.sum(e, axis=1, keepdims=True)
    out_ref[0] = jnp.where(mask, p, jnp.float32(0.0))


def kernel(node_features, node_num_tensor, weight):
    B, L, G = node_features.shape
    lens = node_num_tensor.astype(jnp.int32)
    grid_spec = pltpu.PrefetchScalarGridSpec(
        num_scalar_prefetch=1,
        grid=(B,),
        in_specs=[
            pl.BlockSpec((1, L, G), lambda b, lens_ref: (b, 0, 0)),
            pl.BlockSpec((G, G), lambda b, lens_ref: (0, 0)),
        ],
        out_specs=pl.BlockSpec((1, L, L), lambda b, lens_ref: (b, 0, 0)),
    )
    return pl.pallas_call(
        _edge_att_kernel,
        grid_spec=grid_spec,
        out_shape=jax.ShapeDtypeStruct((B, L, L), jnp.float32),
    )(lens, node_features, weight)


# speedup vs baseline: 2.8967x; 1.0064x over previous
import jax
import jax.numpy as jnp
import numpy as np
from jax.experimental import pallas as pl
from jax.experimental.pallas import tpu as pltpu

WP = 6
WF = 6


def _edge_att_kernel(lens_ref, nf_ref, w_ref, out_ref):
    b = pl.program_id(0)
    nf = nf_ref[0]
    w = w_ref[...]
    att = jnp.dot(nf, w.T, preferred_element_type=jnp.float32)
    scores = jnp.dot(nf, att.T, preferred_element_type=jnp.float32)
    scores = scores * np.float32(1.0 / np.sqrt(200.0))
    L = scores.shape[0]
    j = jax.lax.broadcasted_iota(jnp.int32, (L, L), 0)
    k = jax.lax.broadcasted_iota(jnp.int32, (L, L), 1)
    n = lens_ref[b]
    mask = (k >= j - WP) & (k <= j + WF) & (k < n) & (j < n)
    masked = jnp.where(mask, scores, jnp.float32(-1e9))
    m = jnp.max(masked, axis=1, keepdims=True)
    e = jnp.exp(masked - m)
    p = e / jnp.sum(e, axis=1, keepdims=True)
    out_ref[0] = jnp.where(mask, p, jnp.float32(0.0))


def kernel(node_features, node_num_tensor, weight):
    B, L, G = node_features.shape
    lens = node_num_tensor.astype(jnp.int32)
    grid_spec = pltpu.PrefetchScalarGridSpec(
        num_scalar_prefetch=1,
        grid=(B,),
        in_specs=[
            pl.BlockSpec((1, L, G), lambda b, lens_ref: (b, 0, 0)),
            pl.BlockSpec((G, G), lambda b, lens_ref: (0, 0)),
        ],
        out_specs=pl.BlockSpec((1, L, L), lambda b, lens_ref: (b, 0, 0)),
    )
    return pl.pallas_call(
        _edge_att_kernel,
        grid_spec=grid_spec,
        out_shape=jax.ShapeDtypeStruct((B, L, L), jnp.float32),
        compiler_params=pltpu.CompilerParams(
            dimension_semantics=("parallel",),
        ),
    )(lens, node_features, weight)
